# Initial kernel scaffold; baseline (speedup 1.0000x reference)
import jax
import jax.numpy as jnp
from jax.experimental import pallas as pl


def kernel(output, target, concepts, batch_idx):
    raise NotImplementedError("write your pallas kernel here")


# speedup vs baseline: 1.2332x; 1.2332x over previous
import functools

import jax
import jax.numpy as jnp
from jax import lax
from jax.experimental import pallas as pl
from jax.experimental.pallas import tpu as pltpu
from jax.experimental.pallas import tpu_sc as plsc

B = 1024
V = 100000
TOPK = 100
LS = 0.1
IGN = -100
NUM_STEPS = 100000.0
CONF = 1.0 - LS
TOPK_PCT = 0.05
START = LS / (V - 2)
END = (1.0 - TOPK_PCT) * LS / (V - 2 - TOPK)
STEP = (END - START) / NUM_STEPS
TSTART = LS / (V - 2)
TEND = TOPK_PCT * LS / TOPK
TSTEP = (TEND - TSTART) / NUM_STEPS
IDX0 = V + IGN

GPR = 104
CHUNK = 128
NW = 32
NIDX = B * GPR
PER_W = NIDX // NW
NCH = PER_W // CHUNK


def _sc_gather_body(flat_hbm, idx_hbm, out_hbm, idx_v, vals_v, sem):
    wid = lax.axis_index("s") * 2 + lax.axis_index("c")
    off = wid * PER_W
    pltpu.sync_copy(idx_hbm.at[pl.ds(off, PER_W)], idx_v)
    copies = []
    for j in range(NCH):
        copies.append(
            pltpu.async_copy(flat_hbm.at[idx_v.at[pl.ds(j * CHUNK, CHUNK)]],
                             vals_v.at[pl.ds(j * CHUNK, CHUNK)], sem))
    for c in copies:
        c.wait()
    pltpu.sync_copy(vals_v, out_hbm.at[pl.ds(off, PER_W)])


def _sc_gather(flat, idx):
    run = functools.partial(
        pl.kernel,
        mesh=plsc.VectorSubcoreMesh(core_axis_name="c", subcore_axis_name="s"),
        out_type=jax.ShapeDtypeStruct((NIDX,), jnp.float32),
        scratch_types=[
            pltpu.VMEM((PER_W,), jnp.int32),
            pltpu.VMEM((PER_W,), jnp.float32),
            pltpu.SemaphoreType.DMA,
        ],
    )(_sc_gather_body)
    return run(flat, idx)


RS = 200000
CS = 512
BR = 2000


def _sum_body(x_ref, acc_ref):
    @pl.when(pl.program_id(0) == 0)
    def _():
        acc_ref[0, 0] = 0.0

    acc_ref[0, 0] += jnp.sum(x_ref[...])


def _dense_sum(resh):
    return pl.pallas_call(
        _sum_body,
        grid=(RS // BR,),
        in_specs=[pl.BlockSpec((BR, CS), lambda i: (i, 0))],
        out_specs=pl.BlockSpec((1, 1), lambda i: (0, 0),
                               memory_space=pltpu.SMEM),
        out_shape=jax.ShapeDtypeStruct((1, 1), jnp.float32),
    )(resh)


def _corr_body(g_ref, c_ref, t_ref, consts_ref, psum_ref, out_ref):
    base = consts_ref[0]
    blogb = consts_ref[1]
    k1 = consts_ref[2]
    k2 = consts_ref[3]
    dt = consts_ref[4]
    dconf = consts_ref[5]

    g = g_ref[...]
    c = c_ref[...]
    t = t_ref[...]

    lane = lax.broadcasted_iota(jnp.int32, (B, TOPK), 1)
    isdup = jnp.zeros((B, TOPK), jnp.bool_)
    for j in range(TOPK - 1):
        isdup = isdup | ((c == c[:, j:j + 1]) & (lane > j))

    w = jnp.logical_not(isdup) & (c != t)
    gc = g[:, 2:2 + TOPK]
    corr_c = jnp.sum(jnp.where(w, k2 - dt * gc, 0.0))

    corr_t = jnp.float32(B) * k1 - dconf * jnp.sum(g[:, 0:1])

    hits0 = jnp.sum(jnp.where(c == IDX0, 1, 0), axis=1, keepdims=True)
    untouched = (hits0 == 0) & (t != IDX0)
    corr_0 = jnp.sum(jnp.where(untouched, base * g[:, 1:2] - blogb, 0.0))

    out_ref[0, 0] = (blogb * jnp.float32(B * V) - base * psum_ref[0, 0]
                     + corr_t + corr_c + corr_0)


def _corrections(g, concepts, tcol, consts, psum):
    return pl.pallas_call(
        _corr_body,
        in_specs=[
            pl.BlockSpec((B, GPR), lambda: (0, 0)),
            pl.BlockSpec((B, TOPK), lambda: (0, 0)),
            pl.BlockSpec((B, 1), lambda: (0, 0)),
            pl.BlockSpec(memory_space=pltpu.SMEM),
            pl.BlockSpec(memory_space=pltpu.SMEM),
        ],
        out_specs=pl.BlockSpec(memory_space=pltpu.SMEM),
        out_shape=jax.ShapeDtypeStruct((1, 1), jnp.float32),
    )(g, concepts, tcol, consts, psum)


def kernel(output, target, concepts, batch_idx):
    base = jnp.float32(START + batch_idx * STEP)
    tval = jnp.float32(TSTART + batch_idx * TSTEP)
    blogb = base * jnp.log(base)
    conf = jnp.float32(CONF)
    k1 = conf * jnp.log(conf) - blogb
    k2 = tval * jnp.log(tval) - blogb
    consts = jnp.stack([base, blogb, k1, k2, tval - base, conf - base,
                        jnp.float32(0), jnp.float32(0)])

    tcol = target.reshape(B, 1)
    cols = jnp.concatenate(
        [tcol, jnp.full((B, 1), IDX0, jnp.int32), concepts, tcol, tcol],
        axis=1)
    rowoff = (jnp.arange(B, dtype=jnp.int32) * V)[:, None]
    idx = (cols + rowoff).reshape(NIDX)

    flat = output.reshape(B * V)
    gathered = _sc_gather(flat, idx).reshape(B, GPR)

    psum = _dense_sum(output.reshape(RS, CS))

    total = _corrections(gathered, concepts, tcol, consts, psum)
    return total.reshape(())


# speedup vs baseline: 1.6516x; 1.3393x over previous
import functools

import jax
import jax.numpy as jnp
from jax import lax
from jax.experimental import pallas as pl
from jax.experimental.pallas import tpu as pltpu
from jax.experimental.pallas import tpu_sc as plsc

B = 1024
V = 100000
TOPK = 100
LS = 0.1
IGN = -100
NUM_STEPS = 100000.0
CONF = 1.0 - LS
TOPK_PCT = 0.05
START = LS / (V - 2)
END = (1.0 - TOPK_PCT) * LS / (V - 2 - TOPK)
STEP = (END - START) / NUM_STEPS
TSTART = LS / (V - 2)
TEND = TOPK_PCT * LS / TOPK
TSTEP = (TEND - TSTART) / NUM_STEPS
IDX0 = V + IGN

GPR = 104
CHUNK = 128
NW = 32
NIDX = B * GPR
PER_W = NIDX // NW
NCH = PER_W // CHUNK


def _sc_gather_body(flat_hbm, idx_hbm, out_hbm, idx_v, vals_v, sem):
    wid = lax.axis_index("s") * 2 + lax.axis_index("c")
    off = wid * PER_W
    pltpu.sync_copy(idx_hbm.at[pl.ds(off, PER_W)], idx_v)
    copies = []
    for j in range(NCH):
        copies.append(
            pltpu.async_copy(flat_hbm.at[idx_v.at[pl.ds(j * CHUNK, CHUNK)]],
                             vals_v.at[pl.ds(j * CHUNK, CHUNK)], sem))
    for c in copies:
        c.wait()
    pltpu.sync_copy(vals_v, out_hbm.at[pl.ds(off, PER_W)])


def _sc_gather(flat, idx):
    run = functools.partial(
        pl.kernel,
        mesh=plsc.VectorSubcoreMesh(core_axis_name="c", subcore_axis_name="s"),
        out_type=jax.ShapeDtypeStruct((NIDX,), jnp.float32),
        scratch_types=[
            pltpu.VMEM((PER_W,), jnp.int32),
            pltpu.VMEM((PER_W,), jnp.float32),
            pltpu.SemaphoreType.DMA,
        ],
    )(_sc_gather_body)
    return run(flat, idx)


BW = 4096
NB = -(-V // BW)


def _fold(x):
    parts = [x[:, t * 128:(t + 1) * 128] for t in range(BW // 128)]
    while len(parts) > 1:
        parts = [a + b for a, b in zip(parts[::2], parts[1::2])]
    return parts[0]


def _sum_body(x_ref, out_ref, acc_ref):
    j = pl.program_id(0)

    @pl.when(j == 0)
    def _():
        acc_ref[...] = jnp.zeros_like(acc_ref)

    @pl.when(j < NB - 1)
    def _():
        acc_ref[...] += _fold(x_ref[...])

    @pl.when(j == NB - 1)
    def _():
        col = (j * BW
               + lax.broadcasted_iota(jnp.int32, (B, BW), 1))
        x = jnp.where(col < V, x_ref[...], 0.0)
        acc_ref[...] += _fold(x)
        out_ref[0, 0] = jnp.sum(acc_ref[...])


def _dense_sum(out2d):
    return pl.pallas_call(
        _sum_body,
        grid=(NB,),
        in_specs=[pl.BlockSpec((B, BW), lambda i: (0, i))],
        out_specs=pl.BlockSpec((1, 1), lambda i: (0, 0),
                               memory_space=pltpu.SMEM),
        out_shape=jax.ShapeDtypeStruct((1, 1), jnp.float32),
        scratch_shapes=[pltpu.VMEM((B, 128), jnp.float32)],
    )(out2d)


def _corr_body(g_ref, c_ref, t_ref, consts_ref, psum_ref, out_ref):
    base = consts_ref[0]
    blogb = consts_ref[1]
    k1 = consts_ref[2]
    k2 = consts_ref[3]
    dt = consts_ref[4]
    dconf = consts_ref[5]

    g = g_ref[...]
    c = c_ref[...]
    t = t_ref[...]

    lane = lax.broadcasted_iota(jnp.int32, (B, TOPK), 1)
    isdup = jnp.zeros((B, TOPK), jnp.bool_)
    for j in range(TOPK - 1):
        isdup = isdup | ((c == c[:, j:j + 1]) & (lane > j))

    w = jnp.logical_not(isdup) & (c != t)
    gc = g[:, 2:2 + TOPK]
    corr_c = jnp.sum(jnp.where(w, k2 - dt * gc, 0.0))

    corr_t = jnp.float32(B) * k1 - dconf * jnp.sum(g[:, 0:1])

    hits0 = jnp.sum(jnp.where(c == IDX0, 1, 0), axis=1, keepdims=True)
    untouched = (hits0 == 0) & (t != IDX0)
    corr_0 = jnp.sum(jnp.where(untouched, base * g[:, 1:2] - blogb, 0.0))

    out_ref[0, 0] = (blogb * jnp.float32(B * V) - base * psum_ref[0, 0]
                     + corr_t + corr_c + corr_0)


def _corrections(g, concepts, tcol, consts, psum):
    return pl.pallas_call(
        _corr_body,
        in_specs=[
            pl.BlockSpec((B, GPR), lambda: (0, 0)),
            pl.BlockSpec((B, TOPK), lambda: (0, 0)),
            pl.BlockSpec((B, 1), lambda: (0, 0)),
            pl.BlockSpec(memory_space=pltpu.SMEM),
            pl.BlockSpec(memory_space=pltpu.SMEM),
        ],
        out_specs=pl.BlockSpec(memory_space=pltpu.SMEM),
        out_shape=jax.ShapeDtypeStruct((1, 1), jnp.float32),
    )(g, concepts, tcol, consts, psum)


def kernel(output, target, concepts, batch_idx):
    base = jnp.float32(START + batch_idx * STEP)
    tval = jnp.float32(TSTART + batch_idx * TSTEP)
    blogb = base * jnp.log(base)
    conf = jnp.float32(CONF)
    k1 = conf * jnp.log(conf) - blogb
    k2 = tval * jnp.log(tval) - blogb
    consts = jnp.stack([base, blogb, k1, k2, tval - base, conf - base,
                        jnp.float32(0), jnp.float32(0)])

    tcol = target.reshape(B, 1)
    cols = jnp.concatenate(
        [tcol, jnp.full((B, 1), IDX0, jnp.int32), concepts, tcol, tcol],
        axis=1)
    rowoff = (jnp.arange(B, dtype=jnp.int32) * V)[:, None]
    idx = (cols + rowoff).reshape(NIDX)

    flat = output.reshape(B * V)
    gathered = _sc_gather(flat, idx).reshape(B, GPR)

    psum = _dense_sum(output)

    total = _corrections(gathered, concepts, tcol, consts, psum)
    return total.reshape(())
